# baseline (speedup 1.0000x reference)
import jax
import jax.numpy as jnp
from jax.experimental import pallas as pl
from jax.experimental.pallas import tpu as pltpu

Q, K, D, E, F = 512, 65536, 1024, 4, 2048
KB = 2048
NKB = K // KB
NEG = -1e30
TOPK = 5


def _bf16_round(x):
    b = jax.lax.bitcast_convert_type(x, jnp.uint32)
    r = ((b + jnp.uint32(0x7FFF) + ((b >> 16) & jnp.uint32(1)))
         & jnp.uint32(0xFFFF0000))
    return jax.lax.bitcast_convert_type(r, jnp.float32)


def _moe_kernel(q_ref, wg_ref, w1_ref, w2_ref, out_ref, qn_ref, mask_ref):
    e = pl.program_id(0)

    @pl.when(e == 0)
    def _init():
        q = q_ref[...]
        norm = jnp.sqrt(jnp.sum(q * q, axis=1, keepdims=True))
        qn = q / jnp.maximum(norm, 1e-12)
        qn_ref[...] = qn
        out_ref[0:Q, :] = qn
        out_ref[Q:2 * Q, :] = jnp.zeros((Q, D), jnp.float32)
        logits = jnp.dot(qn, wg_ref[...], preferred_element_type=jnp.float32)
        lane = jax.lax.broadcasted_iota(jnp.int32, logits.shape, 1)
        logits = jnp.where(lane < E, logits, NEG)
        m = jnp.max(logits, axis=1, keepdims=True)
        ex = jnp.exp(logits - m)
        probs = ex / jnp.sum(ex, axis=1, keepdims=True)
        i1 = jnp.argmax(probs, axis=1)[:, None].astype(jnp.int32)
        v1 = jnp.max(probs, axis=1, keepdims=True)
        probs2 = jnp.where(lane == i1, -1.0, probs)
        i2 = jnp.argmax(probs2, axis=1)[:, None].astype(jnp.int32)
        v2 = jnp.max(probs2, axis=1, keepdims=True)
        denom = v1 + v2
        mask = (jnp.where(lane == i1, v1 / denom, 0.0)
                + jnp.where(lane == i2, v2 / denom, 0.0))
        mask_ref[...] = mask

    qn = qn_ref[...]
    h = jnp.maximum(
        jnp.dot(qn, w1_ref[0], preferred_element_type=jnp.float32), 0.0)
    o = jnp.dot(h, w2_ref[0], preferred_element_type=jnp.float32)
    lane = jax.lax.broadcasted_iota(jnp.int32, mask_ref.shape, 1)
    me = jnp.sum(jnp.where(lane == e, mask_ref[...], 0.0), axis=1,
                 keepdims=True)
    out_ref[Q:2 * Q, :] += _bf16_round(me) * _bf16_round(o)


def _moe_stack(queries, wg_pad, w1, w2):
    return pl.pallas_call(
        _moe_kernel,
        grid=(E,),
        in_specs=[
            pl.BlockSpec((Q, D), lambda e: (0, 0)),
            pl.BlockSpec((D, 128), lambda e: (0, 0)),
            pl.BlockSpec((1, D, F), lambda e: (e, 0, 0)),
            pl.BlockSpec((1, F, D), lambda e: (e, 0, 0)),
        ],
        out_specs=pl.BlockSpec((2 * Q, D), lambda e: (0, 0)),
        out_shape=jax.ShapeDtypeStruct((2 * Q, D), jnp.float32),
        scratch_shapes=[
            pltpu.VMEM((Q, D), jnp.float32),
            pltpu.VMEM((Q, 128), jnp.float32),
        ],
        compiler_params=pltpu.CompilerParams(
            dimension_semantics=("arbitrary",)),
    )(queries, wg_pad, w1, w2)


def _topk_kernel(a_ref, keys_ref, vals_ref, idx_ref, sbuf_ref):
    i = pl.program_id(0)

    @pl.when(i == 0)
    def _init():
        vals_ref[...] = jnp.full(vals_ref.shape, NEG, jnp.float32)
        idx_ref[...] = jnp.zeros(idx_ref.shape, jnp.int32)

    @pl.when(i < NKB)
    def _compute():
        kb = keys_ref[...]
        nrm = jnp.sqrt(jnp.sum(kb * kb, axis=1, keepdims=True))
        kn = kb / jnp.maximum(nrm, 1e-12)
        ss = jax.lax.dot_general(
            a_ref[...], kn, (((1,), (1,)), ((), ())),
            preferred_element_type=jnp.float32)
        sbuf_ref[i % 2] = 0.7 * ss[0:Q, :] + 0.3 * ss[Q:2 * Q, :]

    @pl.when(i > 0)
    def _scan():
        s = sbuf_ref[(i - 1) % 2]
        base = (i - 1) * KB
        col = jax.lax.broadcasted_iota(jnp.int32, s.shape, 1)
        bvals, bidx = [], []
        for _ in range(TOPK):
            m = jnp.max(s, axis=1)
            am = jnp.argmax(s, axis=1).astype(jnp.int32)
            bvals.append(m)
            bidx.append(am + base)
            s = jnp.where(col == am[:, None], NEG, s)

        lane = jax.lax.broadcasted_iota(jnp.int32, vals_ref.shape, 1)
        cv = vals_ref[...]
        ci = idx_ref[...]
        for t in range(TOPK):
            cv = jnp.where(lane == 8 + t, bvals[t][:, None], cv)
            ci = jnp.where(lane == 8 + t, bidx[t][:, None], ci)
        nv = jnp.full(vals_ref.shape, NEG, jnp.float32)
        ni = jnp.zeros(idx_ref.shape, jnp.int32)
        for t in range(TOPK):
            m = jnp.max(cv, axis=1)
            am = jnp.argmax(cv, axis=1).astype(jnp.int32)[:, None]
            sel = jnp.sum(jnp.where(lane == am, ci, 0), axis=1)
            nv = jnp.where(lane == t, m[:, None], nv)
            ni = jnp.where(lane == t, sel[:, None], ni)
            cv = jnp.where(lane == am, NEG, cv)
        vals_ref[...] = nv
        idx_ref[...] = ni


def _score_topk(a, keys):
    return pl.pallas_call(
        _topk_kernel,
        grid=(NKB + 1,),
        in_specs=[
            pl.BlockSpec((2 * Q, D), lambda i: (0, 0)),
            pl.BlockSpec((KB, D), lambda i: (jnp.minimum(i, NKB - 1), 0)),
        ],
        out_specs=[
            pl.BlockSpec((Q, 128), lambda i: (0, 0)),
            pl.BlockSpec((Q, 128), lambda i: (0, 0)),
        ],
        out_shape=[
            jax.ShapeDtypeStruct((Q, 128), jnp.float32),
            jax.ShapeDtypeStruct((Q, 128), jnp.int32),
        ],
        scratch_shapes=[pltpu.VMEM((2, Q, KB), jnp.float32)],
        compiler_params=pltpu.CompilerParams(
            dimension_semantics=("arbitrary",)),
    )(a, keys)


def kernel(queries, keys, Wg, W1, W2, top_k):
    wg_pad = jnp.zeros((D, 128), jnp.float32).at[:, :E].set(Wg)
    a = _moe_stack(queries, wg_pad, W1, W2)
    vals, idx = _score_topk(a, keys)
    return vals[:, :TOPK], idx[:, :TOPK]


# speedup vs baseline: 1.1652x; 1.1652x over previous
import jax
import jax.numpy as jnp
from jax.experimental import pallas as pl
from jax.experimental.pallas import tpu as pltpu

Q, K, D, E, F = 512, 65536, 1024, 4, 2048
KB = 2048
NKB = K // KB
NEG = -1e30
TOPK = 5


def _bf16_round(x):
    b = jax.lax.bitcast_convert_type(x, jnp.uint32)
    r = ((b + jnp.uint32(0x7FFF) + ((b >> 16) & jnp.uint32(1)))
         & jnp.uint32(0xFFFF0000))
    return jax.lax.bitcast_convert_type(r, jnp.float32)


def _moe_kernel(q_ref, wg_ref, w1_ref, w2_ref, out_ref, qn_ref, mask_ref):
    e = pl.program_id(0)

    @pl.when(e == 0)
    def _init():
        q = q_ref[...]
        norm = jnp.sqrt(jnp.sum(q * q, axis=1, keepdims=True))
        qn = q / jnp.maximum(norm, 1e-12)
        qn_ref[...] = qn
        out_ref[0:Q, :] = qn
        out_ref[Q:2 * Q, :] = jnp.zeros((Q, D), jnp.float32)
        logits = jnp.dot(qn, wg_ref[...], preferred_element_type=jnp.float32)
        lane = jax.lax.broadcasted_iota(jnp.int32, logits.shape, 1)
        logits = jnp.where(lane < E, logits, NEG)
        m = jnp.max(logits, axis=1, keepdims=True)
        ex = jnp.exp(logits - m)
        probs = ex / jnp.sum(ex, axis=1, keepdims=True)
        i1 = jnp.argmax(probs, axis=1)[:, None].astype(jnp.int32)
        v1 = jnp.max(probs, axis=1, keepdims=True)
        probs2 = jnp.where(lane == i1, -1.0, probs)
        i2 = jnp.argmax(probs2, axis=1)[:, None].astype(jnp.int32)
        v2 = jnp.max(probs2, axis=1, keepdims=True)
        denom = v1 + v2
        mask = (jnp.where(lane == i1, v1 / denom, 0.0)
                + jnp.where(lane == i2, v2 / denom, 0.0))
        mask_ref[...] = mask

    qn = qn_ref[...]
    h = jnp.maximum(
        jnp.dot(qn, w1_ref[0], preferred_element_type=jnp.float32), 0.0)
    o = jnp.dot(h, w2_ref[0], preferred_element_type=jnp.float32)
    lane = jax.lax.broadcasted_iota(jnp.int32, mask_ref.shape, 1)
    me = jnp.sum(jnp.where(lane == e, mask_ref[...], 0.0), axis=1,
                 keepdims=True)
    out_ref[Q:2 * Q, :] += _bf16_round(me) * _bf16_round(o)


def _moe_stack(queries, wg_pad, w1, w2):
    return pl.pallas_call(
        _moe_kernel,
        grid=(E,),
        in_specs=[
            pl.BlockSpec((Q, D), lambda e: (0, 0)),
            pl.BlockSpec((D, 128), lambda e: (0, 0)),
            pl.BlockSpec((1, D, F), lambda e: (e, 0, 0)),
            pl.BlockSpec((1, F, D), lambda e: (e, 0, 0)),
        ],
        out_specs=pl.BlockSpec((2 * Q, D), lambda e: (0, 0)),
        out_shape=jax.ShapeDtypeStruct((2 * Q, D), jnp.float32),
        scratch_shapes=[
            pltpu.VMEM((Q, D), jnp.float32),
            pltpu.VMEM((Q, 128), jnp.float32),
        ],
        compiler_params=pltpu.CompilerParams(
            dimension_semantics=("arbitrary",)),
    )(queries, wg_pad, w1, w2)


def _dot_block(a_ref, keys_ref, out_sref):
    kb = keys_ref[...]
    nrm = jnp.sqrt(jnp.sum(kb * kb, axis=1, keepdims=True))
    kn = kb / jnp.maximum(nrm, 1e-12)
    ss = jax.lax.dot_general(
        a_ref[...], kn, (((1,), (1,)), ((), ())),
        preferred_element_type=jnp.float32)
    out_sref[...] = 0.7 * ss[0:Q, :] + 0.3 * ss[Q:2 * Q, :]


def _scan_block(s, base, vals_ref, idx_ref):
    col = jax.lax.broadcasted_iota(jnp.int32, s.shape, 1)
    bvals, bidx = [], []
    for _ in range(TOPK):
        m = jnp.max(s, axis=1)
        am = jnp.argmax(s, axis=1).astype(jnp.int32)
        bvals.append(m)
        bidx.append(am + base)
        s = jnp.where(col == am[:, None], NEG, s)

    lane = jax.lax.broadcasted_iota(jnp.int32, vals_ref.shape, 1)
    cv = vals_ref[...]
    ci = idx_ref[...]
    for t in range(TOPK):
        cv = jnp.where(lane == 8 + t, bvals[t][:, None], cv)
        ci = jnp.where(lane == 8 + t, bidx[t][:, None], ci)
    nv = jnp.full(vals_ref.shape, NEG, jnp.float32)
    ni = jnp.zeros(idx_ref.shape, jnp.int32)
    for t in range(TOPK):
        m = jnp.max(cv, axis=1)
        am = jnp.argmax(cv, axis=1).astype(jnp.int32)[:, None]
        sel = jnp.sum(jnp.where(lane == am, ci, 0), axis=1)
        nv = jnp.where(lane == t, m[:, None], nv)
        ni = jnp.where(lane == t, sel[:, None], ni)
        cv = jnp.where(lane == am, NEG, cv)
    vals_ref[...] = nv
    idx_ref[...] = ni


def _topk_kernel(a_ref, keys_ref, vals_ref, idx_ref, sa_ref, sb_ref):
    i = pl.program_id(0)

    @pl.when(i == 0)
    def _init():
        vals_ref[...] = jnp.full(vals_ref.shape, NEG, jnp.float32)
        idx_ref[...] = jnp.zeros(idx_ref.shape, jnp.int32)
        sb_ref[...] = jnp.full(sb_ref.shape, NEG, jnp.float32)

    base = (i - 1) * KB

    @pl.when(i % 2 == 0)
    def _even():
        _dot_block(a_ref, keys_ref, sa_ref)
        _scan_block(sb_ref[...], base, vals_ref, idx_ref)

    @pl.when(i % 2 == 1)
    def _odd():
        _dot_block(a_ref, keys_ref, sb_ref)
        _scan_block(sa_ref[...], base, vals_ref, idx_ref)


def _score_topk(a, keys):
    return pl.pallas_call(
        _topk_kernel,
        grid=(NKB + 1,),
        in_specs=[
            pl.BlockSpec((2 * Q, D), lambda i: (0, 0)),
            pl.BlockSpec((KB, D), lambda i: (jnp.minimum(i, NKB - 1), 0)),
        ],
        out_specs=[
            pl.BlockSpec((Q, 128), lambda i: (0, 0)),
            pl.BlockSpec((Q, 128), lambda i: (0, 0)),
        ],
        out_shape=[
            jax.ShapeDtypeStruct((Q, 128), jnp.float32),
            jax.ShapeDtypeStruct((Q, 128), jnp.int32),
        ],
        scratch_shapes=[pltpu.VMEM((Q, KB), jnp.float32),
                        pltpu.VMEM((Q, KB), jnp.float32)],
        compiler_params=pltpu.CompilerParams(
            dimension_semantics=("arbitrary",)),
    )(a, keys)


def kernel(queries, keys, Wg, W1, W2, top_k):
    wg_pad = jnp.zeros((D, 128), jnp.float32).at[:, :E].set(Wg)
    a = _moe_stack(queries, wg_pad, W1, W2)
    vals, idx = _score_topk(a, keys)
    return vals[:, :TOPK], idx[:, :TOPK]
